# baseline (speedup 1.0000x reference)
import functools
import jax
import jax.numpy as jnp
from jax import lax
from jax.experimental import pallas as pl
from jax.experimental.pallas import tpu as pltpu
from jax.experimental.pallas import tpu_sc as plsc


def _make_sc_gather(N, D, n_per, CH):
    info = plsc.get_sparse_core_info()
    NC = info.num_cores

    mesh = plsc.VectorSubcoreMesh(core_axis_name="c", subcore_axis_name="s")

    @functools.partial(
        pl.kernel,
        mesh=mesh,
        out_type=jax.ShapeDtypeStruct((N, D), jnp.float32),
        scratch_types=[
            pltpu.VMEM((n_per,), jnp.int32),
            pltpu.VMEM((CH, D), jnp.float32),
            pltpu.SemaphoreType.DMA,
        ],
    )
    def gather_kernel(table_hbm, idx_hbm, out_hbm, idx_v, buf_v, sem):
        c = lax.axis_index("c")
        s = lax.axis_index("s")
        wid = s * NC + c
        base = wid * n_per
        pltpu.sync_copy(idx_hbm.at[pl.ds(base, n_per)], idx_v)

        def body(k, carry):
            off = k * CH
            cp = pltpu.async_copy(
                table_hbm.at[idx_v.at[pl.ds(off, CH)]], buf_v, sem
            )
            cp.wait()
            pltpu.sync_copy(buf_v, out_hbm.at[pl.ds(base + off, CH)])
            return carry

        lax.fori_loop(0, n_per // CH, body, 0)

    return gather_kernel


def _mlp_body(g1_ref, g2_ref, w1a_ref, w1b_ref, b1_ref, w2_ref, i_ref, out_ref):
    f1 = g1_ref[0, 0, 0]
    f2 = g2_ref[0, 0, 0]
    t = (
        jnp.dot(f1, w1a_ref[...], preferred_element_type=jnp.float32)
        + jnp.dot(f2, w1b_ref[...], preferred_element_type=jnp.float32)
        + b1_ref[...]
    )
    h = jnp.maximum(t, 0.0)
    o = jnp.dot(h, w2_ref[...], preferred_element_type=jnp.float32)
    orow = lax.dot_general(
        o, i_ref[...], (((0,), (0,)), ((), ())),
        preferred_element_type=jnp.float32,
    )
    out_ref[0, 0] = orow


def _mlp_call(gathered, W1a, W1b, b1r, W2, I, B, H, D, interpret=False):
    grid = (B, H)
    return pl.pallas_call(
        _mlp_body,
        grid=grid,
        in_specs=[
            pl.BlockSpec((1, 1, 1, H, D), lambda b, i: (0, b, i, 0, 0)),
            pl.BlockSpec((1, 1, 1, H, D), lambda b, i: (1, b, i, 0, 0)),
            pl.BlockSpec((D, 64), lambda b, i: (0, 0)),
            pl.BlockSpec((D, 64), lambda b, i: (0, 0)),
            pl.BlockSpec((1, 64), lambda b, i: (0, 0)),
            pl.BlockSpec((64, 1), lambda b, i: (0, 0)),
            pl.BlockSpec((H, H), lambda b, i: (0, 0)),
        ],
        out_specs=pl.BlockSpec((1, 1, 1, H), lambda b, i: (b, i, 0, 0)),
        out_shape=jax.ShapeDtypeStruct((B, H, 1, H), jnp.float32),
        interpret=interpret,
    )(gathered, gathered, W1a, W1b, b1r, W2, I)


def kernel(pair_repr, helix_ends_list, W1, b1, W2, b2):
    B, L, _, D = pair_repr.shape
    H = helix_ends_list.shape[1]
    i5 = helix_ends_list[:, :, 1]
    i3 = helix_ends_list[:, :, 2]

    boff = (jnp.arange(B, dtype=jnp.int32) * (L * L))[:, None, None]
    idx1 = boff + i5[:, :, None] * L + i5[:, None, :]
    idx2 = boff + i3[:, :, None] * L + i3[:, None, :]
    idx_all = jnp.stack([idx1, idx2], axis=0)

    table = pair_repr.reshape(B * L * L, D)
    N = 2 * B * H * H
    n_per = N // 32
    gather_fn = _make_sc_gather(N, D, n_per, 128)
    gathered = gather_fn(table, idx_all.reshape(-1)).reshape(2, B, H, H, D)

    W1a = W1[:D]
    W1b = W1[D:]
    I = jnp.eye(H, dtype=jnp.float32)

    out = _mlp_call(gathered, W1a, W1b, b1.reshape(1, 64), W2, I, B, H, D)
    return out.reshape(B, H, H) + b2[0]


# speedup vs baseline: 2.3948x; 2.3948x over previous
import functools
import jax
import jax.numpy as jnp
from jax import lax
from jax.experimental import pallas as pl
from jax.experimental.pallas import tpu as pltpu
from jax.experimental.pallas import tpu_sc as plsc


def _make_sc_gather(N, D, n_per, CH):
    info = plsc.get_sparse_core_info()
    NC = info.num_cores

    mesh = plsc.VectorSubcoreMesh(core_axis_name="c", subcore_axis_name="s")

    @functools.partial(
        pl.kernel,
        mesh=mesh,
        out_type=jax.ShapeDtypeStruct((N, D), jnp.float32),
        scratch_types=[
            pltpu.VMEM((n_per,), jnp.int32),
            pltpu.VMEM((CH, D), jnp.float32),
            pltpu.SemaphoreType.DMA,
        ],
    )
    def gather_kernel(table_hbm, idx_hbm, out_hbm, idx_v, buf_v, sem):
        c = lax.axis_index("c")
        s = lax.axis_index("s")
        wid = s * NC + c
        base = wid * n_per
        pltpu.sync_copy(idx_hbm.at[pl.ds(base, n_per)], idx_v)

        def body(k, carry):
            off = k * CH
            cp = pltpu.async_copy(
                table_hbm.at[idx_v.at[pl.ds(off, CH)]], buf_v, sem
            )
            cp.wait()
            pltpu.sync_copy(buf_v, out_hbm.at[pl.ds(base + off, CH)])
            return carry

        lax.fori_loop(0, n_per // CH, body, 0)

    return gather_kernel


_IB = 8


def _mlp_body(g1_ref, g2_ref, w1a_ref, w1b_ref, b1_ref, w2_ref, i_ref, out_ref):
    f1 = g1_ref[0, 0, 0]
    f2 = g2_ref[0, 0, 0]
    t = (
        jnp.dot(f1, w1a_ref[...], preferred_element_type=jnp.float32)
        + jnp.dot(f2, w1b_ref[...], preferred_element_type=jnp.float32)
        + b1_ref[...]
    )
    h = jnp.maximum(t, 0.0)
    o = jnp.dot(h, w2_ref[...], preferred_element_type=jnp.float32)
    orow = lax.dot_general(
        o, i_ref[...], (((0,), (0,)), ((), ())),
        preferred_element_type=jnp.float32,
    )
    out_ref[0, 0] = orow


def _mlp_call(gathered, W1a, W1b, b1r, W2, I, B, H, D, interpret=False):
    n = _IB * H
    grid = (B, H // _IB)
    return pl.pallas_call(
        _mlp_body,
        grid=grid,
        in_specs=[
            pl.BlockSpec((1, 1, 1, n, D), lambda b, i: (0, b, i, 0, 0)),
            pl.BlockSpec((1, 1, 1, n, D), lambda b, i: (1, b, i, 0, 0)),
            pl.BlockSpec((D, 64), lambda b, i: (0, 0)),
            pl.BlockSpec((D, 64), lambda b, i: (0, 0)),
            pl.BlockSpec((1, 64), lambda b, i: (0, 0)),
            pl.BlockSpec((64, 1), lambda b, i: (0, 0)),
            pl.BlockSpec((n, n), lambda b, i: (0, 0)),
        ],
        out_specs=pl.BlockSpec((1, 1, 1, n), lambda b, i: (b, i, 0, 0)),
        out_shape=jax.ShapeDtypeStruct((B, H // _IB, 1, n), jnp.float32),
        interpret=interpret,
    )(gathered.reshape(2, B, H // _IB, _IB * H, D),
      gathered.reshape(2, B, H // _IB, _IB * H, D),
      W1a, W1b, b1r, W2, I)


def kernel(pair_repr, helix_ends_list, W1, b1, W2, b2):
    B, L, _, D = pair_repr.shape
    H = helix_ends_list.shape[1]
    i5 = helix_ends_list[:, :, 1]
    i3 = helix_ends_list[:, :, 2]

    boff = (jnp.arange(B, dtype=jnp.int32) * (L * L))[:, None, None]
    idx1 = boff + i5[:, :, None] * L + i5[:, None, :]
    idx2 = boff + i3[:, :, None] * L + i3[:, None, :]
    idx_all = jnp.stack([idx1, idx2], axis=0)

    table = pair_repr.reshape(B * L * L, D)
    N = 2 * B * H * H
    n_per = N // 32
    gather_fn = _make_sc_gather(N, D, n_per, 128)
    gathered = gather_fn(table, idx_all.reshape(-1)).reshape(2, B, H, H, D)

    W1a = W1[:D]
    W1b = W1[D:]
    I = jnp.eye(_IB * H, dtype=jnp.float32)

    out = _mlp_call(gathered, W1a, W1b, b1.reshape(1, 64), W2, I, B, H, D)
    return out.reshape(B, H, H) + b2[0]


# speedup vs baseline: 2.5355x; 1.0587x over previous
import functools
import jax
import jax.numpy as jnp
from jax import lax
from jax.experimental import pallas as pl
from jax.experimental.pallas import tpu as pltpu
from jax.experimental.pallas import tpu_sc as plsc


def _make_sc_gather(N, D, n_per, CH):
    info = plsc.get_sparse_core_info()
    NC = info.num_cores

    mesh = plsc.VectorSubcoreMesh(core_axis_name="c", subcore_axis_name="s")

    @functools.partial(
        pl.kernel,
        mesh=mesh,
        out_type=jax.ShapeDtypeStruct((N, 2 * D), jnp.float32),
        scratch_types=[
            pltpu.VMEM((n_per,), jnp.int32),
            pltpu.VMEM((n_per,), jnp.int32),
            pltpu.VMEM((CH, D), jnp.float32),
            pltpu.VMEM((CH, D), jnp.float32),
            pltpu.SemaphoreType.DMA,
        ],
    )
    def gather_kernel(table_hbm, idx_hbm, out_hbm, idx1_v, idx2_v, buf1_v,
                      buf2_v, sem):
        c = lax.axis_index("c")
        s = lax.axis_index("s")
        wid = s * NC + c
        base = wid * n_per
        pltpu.sync_copy(idx_hbm.at[0, pl.ds(base, n_per)], idx1_v)
        pltpu.sync_copy(idx_hbm.at[1, pl.ds(base, n_per)], idx2_v)

        def body(k, carry):
            off = k * CH
            cp1 = pltpu.async_copy(
                table_hbm.at[idx1_v.at[pl.ds(off, CH)]], buf1_v, sem
            )
            cp2 = pltpu.async_copy(
                table_hbm.at[idx2_v.at[pl.ds(off, CH)]], buf2_v, sem
            )
            cp1.wait()
            cp2.wait()
            pltpu.sync_copy(
                buf1_v, out_hbm.at[pl.ds(base + off, CH), pl.ds(0, D)]
            )
            pltpu.sync_copy(
                buf2_v, out_hbm.at[pl.ds(base + off, CH), pl.ds(D, D)]
            )
            return carry

        lax.fori_loop(0, n_per // CH, body, 0)

    return gather_kernel


_IB = 8


def _mlp_body(g_ref, w1_ref, b1_ref, w2_ref, i_ref, out_ref):
    f = g_ref[0, 0]
    t = jnp.dot(f, w1_ref[...], preferred_element_type=jnp.float32) + b1_ref[...]
    h = jnp.maximum(t, 0.0)
    o = jnp.dot(h, w2_ref[...], preferred_element_type=jnp.float32)
    orow = lax.dot_general(
        o, i_ref[...], (((0,), (0,)), ((), ())),
        preferred_element_type=jnp.float32,
    )
    out_ref[0, 0] = orow


def _mlp_call(gathered, W1, b1r, W2, I, B, H, D, interpret=False):
    n = _IB * H
    grid = (B, H // _IB)
    return pl.pallas_call(
        _mlp_body,
        grid=grid,
        in_specs=[
            pl.BlockSpec((1, 1, n, 2 * D), lambda b, i: (b, i, 0, 0)),
            pl.BlockSpec((2 * D, 64), lambda b, i: (0, 0)),
            pl.BlockSpec((1, 64), lambda b, i: (0, 0)),
            pl.BlockSpec((64, 1), lambda b, i: (0, 0)),
            pl.BlockSpec((n, n), lambda b, i: (0, 0)),
        ],
        out_specs=pl.BlockSpec((1, 1, 1, n), lambda b, i: (b, i, 0, 0)),
        out_shape=jax.ShapeDtypeStruct((B, H // _IB, 1, n), jnp.float32),
        interpret=interpret,
    )(gathered.reshape(B, H // _IB, n, 2 * D), W1, b1r, W2, I)


def kernel(pair_repr, helix_ends_list, W1, b1, W2, b2):
    B, L, _, D = pair_repr.shape
    H = helix_ends_list.shape[1]
    i5 = helix_ends_list[:, :, 1]
    i3 = helix_ends_list[:, :, 2]

    boff = (jnp.arange(B, dtype=jnp.int32) * (L * L))[:, None, None]
    idx1 = boff + i5[:, :, None] * L + i5[:, None, :]
    idx2 = boff + i3[:, :, None] * L + i3[:, None, :]
    idx_all = jnp.stack([idx1.reshape(-1), idx2.reshape(-1)], axis=0)

    table = pair_repr.reshape(B * L * L, D)
    N = B * H * H
    n_per = N // 32
    gather_fn = _make_sc_gather(N, D, n_per, 128)
    gathered = gather_fn(table, idx_all)

    I = jnp.eye(_IB * H, dtype=jnp.float32)
    out = _mlp_call(gathered, W1, b1.reshape(1, 64), W2, I, B, H, D)
    return out.reshape(B, H, H) + b2[0]


# speedup vs baseline: 2.6716x; 1.0537x over previous
import functools
import jax
import jax.numpy as jnp
from jax import lax
from jax.experimental import pallas as pl
from jax.experimental.pallas import tpu as pltpu
from jax.experimental.pallas import tpu_sc as plsc


def _make_sc_gather(N, D, n_per, CH):
    info = plsc.get_sparse_core_info()
    NC = info.num_cores

    mesh = plsc.VectorSubcoreMesh(core_axis_name="c", subcore_axis_name="s")

    @functools.partial(
        pl.kernel,
        mesh=mesh,
        out_type=jax.ShapeDtypeStruct((N, 2 * D), jnp.float32),
        scratch_types=[
            pltpu.VMEM((n_per,), jnp.int32),
            pltpu.VMEM((n_per,), jnp.int32),
            pltpu.VMEM((2, CH, D), jnp.float32),
            pltpu.VMEM((2, CH, D), jnp.float32),
            pltpu.SemaphoreType.DMA,
            pltpu.SemaphoreType.DMA,
        ],
    )
    def gather_kernel(table_hbm, idx_hbm, out_hbm, idx1_v, idx2_v, buf1_v,
                      buf2_v, gsem, wsem):
        c = lax.axis_index("c")
        s = lax.axis_index("s")
        wid = s * NC + c
        base = wid * n_per
        pltpu.sync_copy(idx_hbm.at[0, pl.ds(base, n_per)], idx1_v)
        pltpu.sync_copy(idx_hbm.at[1, pl.ds(base, n_per)], idx2_v)

        nch = n_per // CH
        gath = [None, None]
        writes = [None, None]

        def start_gather(k):
            sl = k % 2
            off = k * CH
            g1 = pltpu.async_copy(
                table_hbm.at[idx1_v.at[pl.ds(off, CH)]], buf1_v.at[sl], gsem
            )
            g2 = pltpu.async_copy(
                table_hbm.at[idx2_v.at[pl.ds(off, CH)]], buf2_v.at[sl], gsem
            )
            gath[sl] = (g1, g2)

        def start_write(k):
            sl = k % 2
            off = k * CH
            gath[sl][0].wait()
            gath[sl][1].wait()
            w1 = pltpu.async_copy(
                buf1_v.at[sl], out_hbm.at[pl.ds(base + off, CH), pl.ds(0, D)],
                wsem,
            )
            w2 = pltpu.async_copy(
                buf2_v.at[sl], out_hbm.at[pl.ds(base + off, CH), pl.ds(D, D)],
                wsem,
            )
            writes[sl] = (w1, w2)

        for k in range(nch):
            sl = k % 2
            if writes[sl] is not None:
                writes[sl][0].wait()
                writes[sl][1].wait()
                writes[sl] = None
            start_gather(k)
            if k > 0:
                start_write(k - 1)
        start_write(nch - 1)
        for sl in range(2):
            if writes[sl] is not None:
                writes[sl][0].wait()
                writes[sl][1].wait()

    return gather_kernel


_IB = 8


def _mlp_body(g_ref, w1_ref, b1_ref, w2_ref, i_ref, out_ref):
    f = g_ref[0, 0]
    t = jnp.dot(f, w1_ref[...], preferred_element_type=jnp.float32) + b1_ref[...]
    h = jnp.maximum(t, 0.0)
    o = jnp.dot(h, w2_ref[...], preferred_element_type=jnp.float32)
    orow = lax.dot_general(
        o, i_ref[...], (((0,), (0,)), ((), ())),
        preferred_element_type=jnp.float32,
    )
    out_ref[0, 0] = orow


def _mlp_call(gathered, W1, b1r, W2, I, B, H, D, interpret=False):
    n = _IB * H
    grid = (B, H // _IB)
    return pl.pallas_call(
        _mlp_body,
        grid=grid,
        in_specs=[
            pl.BlockSpec((1, 1, n, 2 * D), lambda b, i: (b, i, 0, 0)),
            pl.BlockSpec((2 * D, 64), lambda b, i: (0, 0)),
            pl.BlockSpec((1, 64), lambda b, i: (0, 0)),
            pl.BlockSpec((64, 1), lambda b, i: (0, 0)),
            pl.BlockSpec((n, n), lambda b, i: (0, 0)),
        ],
        out_specs=pl.BlockSpec((1, 1, 1, n), lambda b, i: (b, i, 0, 0)),
        out_shape=jax.ShapeDtypeStruct((B, H // _IB, 1, n), jnp.float32),
        interpret=interpret,
    )(gathered.reshape(B, H // _IB, n, 2 * D), W1, b1r, W2, I)


def kernel(pair_repr, helix_ends_list, W1, b1, W2, b2):
    B, L, _, D = pair_repr.shape
    H = helix_ends_list.shape[1]
    i5 = helix_ends_list[:, :, 1]
    i3 = helix_ends_list[:, :, 2]

    boff = (jnp.arange(B, dtype=jnp.int32) * (L * L))[:, None, None]
    idx1 = boff + i5[:, :, None] * L + i5[:, None, :]
    idx2 = boff + i3[:, :, None] * L + i3[:, None, :]
    idx_all = jnp.stack([idx1.reshape(-1), idx2.reshape(-1)], axis=0)

    table = pair_repr.reshape(B * L * L, D)
    N = B * H * H
    n_per = N // 32
    gather_fn = _make_sc_gather(N, D, n_per, 128)
    gathered = gather_fn(table, idx_all)

    I = jnp.eye(_IB * H, dtype=jnp.float32)
    out = _mlp_call(gathered, W1, b1.reshape(1, 64), W2, I, B, H, D)
    return out.reshape(B, H, H) + b2[0]
